# baseline (device time: 159978 ns/iter reference)
import jax
import jax.numpy as jnp
from jax import lax
from jax.experimental import pallas as pl
from jax.experimental.pallas import tpu as pltpu

N_DEV = 4
N_RS = N_DEV - 1
N_HOP = 2 * N_RS
N_SUB = 2
RINGS = ("a", "b")


def kernel(x):
    m_per, n = x.shape
    half = m_per // 2
    chunk = half // N_DEV
    sub = chunk // N_SUB

    def send_chunk(d, t, ring):
        if t < N_RS:
            return (d - t) % N_DEV if ring == "a" else (d + t) % N_DEV
        s = t - N_RS
        return (d + 1 - s) % N_DEV if ring == "a" else (d - 1 + s) % N_DEV

    def recv_chunk(d, t, ring):
        if t < N_RS:
            return (d - t - 1) % N_DEV if ring == "a" else (d + t + 1) % N_DEV
        s = t - N_RS
        return (d - s) % N_DEV if ring == "a" else (d + s) % N_DEV

    def body(x_ref, out_hbm, vout, rs_a, rs_b,
             send_sems_a, recv_sems_a, send_sems_b, recv_sems_b,
             outcopy_sems):
        d = lax.axis_index("i")
        left = (d - 1) % N_DEV
        right = (d + 1) % N_DEV

        barrier_sem = pltpu.get_barrier_semaphore()
        for nbr in (left, right):
            pl.semaphore_signal(
                barrier_sem, inc=1,
                device_id=(nbr,), device_id_type=pl.DeviceIdType.MESH,
            )
        pl.semaphore_wait(barrier_sem, 2)

        def rows(t, s, ring, which):
            c = (send_chunk if which == "send" else recv_chunk)(d, t, ring)
            base = 0 if ring == "a" else half
            return pl.ds(base + c * chunk + s * sub, sub)

        def ring_parts(ring):
            if ring == "a":
                return rs_a, send_sems_a, recv_sems_a, right, left
            return rs_b, send_sems_b, recv_sems_b, left, right

        def make_send(t, s, ring):
            rs, send_sems, recv_sems, dst_nbr, _ = ring_parts(ring)
            r = rows(t, s, ring, "send")
            src = x_ref.at[r, :] if t == 0 else vout.at[r, :]
            dst = rs.at[t, s] if t < N_RS else vout.at[r, :]
            return pltpu.make_async_remote_copy(
                src_ref=src, dst_ref=dst,
                send_sem=send_sems.at[t, s], recv_sem=recv_sems.at[t, s],
                device_id=(dst_nbr,), device_id_type=pl.DeviceIdType.MESH,
            )

        local = []
        pending = []

        for s in range(N_SUB):
            for ring in RINGS:
                r = make_send(0, s, ring)
                r.start()
                pending.append(r)

        for t in range(N_HOP):
            for s in range(N_SUB):
                for ring in RINGS:
                    rs, _, recv_sems, _, src_nbr = ring_parts(ring)
                    rr = rows(t, s, ring, "recv")
                    dst = rs.at[t, s] if t < N_RS else vout.at[rr, :]
                    recv = pltpu.make_async_remote_copy(
                        src_ref=dst, dst_ref=dst,
                        send_sem=recv_sems.at[t, s], recv_sem=recv_sems.at[t, s],
                        device_id=(src_nbr,), device_id_type=pl.DeviceIdType.MESH,
                    )
                    recv.wait_recv()
                    if t < N_RS:
                        vout[rr, :] = x_ref[rr, :] + rs[t, s]
                    if t < N_HOP - 1:
                        r = make_send(t + 1, s, ring)
                        r.start()
                        pending.append(r)
                if t >= N_RS - 1:
                    for ri, ring in enumerate(RINGS):
                        rr = rows(t, s, ring, "recv")
                        cp = pltpu.make_async_copy(
                            vout.at[rr, :], out_hbm.at[rr, :],
                            outcopy_sems.at[ri, t - N_RS + 1, s],
                        )
                        cp.start()
                        local.append(cp)

        for r in pending:
            r.wait_send()
        for cp in local:
            cp.wait()

    return pl.pallas_call(
        body,
        out_shape=jax.ShapeDtypeStruct((m_per, n), x.dtype),
        in_specs=[pl.BlockSpec(memory_space=pltpu.VMEM)],
        out_specs=pl.BlockSpec(memory_space=pl.ANY),
        scratch_shapes=[
            pltpu.VMEM((m_per, n), x.dtype),
            pltpu.VMEM((N_RS, N_SUB, sub, n), x.dtype),
            pltpu.VMEM((N_RS, N_SUB, sub, n), x.dtype),
            pltpu.SemaphoreType.DMA((N_HOP, N_SUB)),
            pltpu.SemaphoreType.DMA((N_HOP, N_SUB)),
            pltpu.SemaphoreType.DMA((N_HOP, N_SUB)),
            pltpu.SemaphoreType.DMA((N_HOP, N_SUB)),
            pltpu.SemaphoreType.DMA((2, N_RS + 1, N_SUB)),
        ],
        compiler_params=pltpu.CompilerParams(
            collective_id=0, vmem_limit_bytes=60 * 1024 * 1024,
        ),
    )(x)


# device time: 97127 ns/iter; 1.6471x vs baseline; 1.6471x over previous
import jax
import jax.numpy as jnp
from jax import lax
from jax.experimental import pallas as pl
from jax.experimental.pallas import tpu as pltpu

N_DEV = 4
N_RS = N_DEV - 1
N_HOP = 2 * N_RS
N_SUB = 2
RINGS = ("a", "b")


def kernel(x):
    m_per, n = x.shape
    half = m_per // 2
    chunk = half // N_DEV
    sub = chunk // N_SUB

    def send_chunk(d, t, ring):
        if t < N_RS:
            return (d - t) % N_DEV if ring == "a" else (d + t) % N_DEV
        s = t - N_RS
        return (d + 1 - s) % N_DEV if ring == "a" else (d - 1 + s) % N_DEV

    def recv_chunk(d, t, ring):
        if t < N_RS:
            return (d - t - 1) % N_DEV if ring == "a" else (d + t + 1) % N_DEV
        s = t - N_RS
        return (d - s) % N_DEV if ring == "a" else (d + s) % N_DEV

    def body(x_ref, out_ref, rs_a, rs_b, ag_a, ag_b, sb_a, sb_b,
             send_sems_a, recv_sems_a, send_sems_b, recv_sems_b):
        d = lax.axis_index("i")
        left = (d - 1) % N_DEV
        right = (d + 1) % N_DEV

        barrier_sem = pltpu.get_barrier_semaphore()
        for nbr in (left, right):
            pl.semaphore_signal(
                barrier_sem, inc=1,
                device_id=(nbr,), device_id_type=pl.DeviceIdType.MESH,
            )
        pl.semaphore_wait(barrier_sem, 2)

        def rows(t, s, ring, which):
            c = (send_chunk if which == "send" else recv_chunk)(d, t, ring)
            base = 0 if ring == "a" else half
            return pl.ds(base + c * chunk + s * sub, sub)

        def ring_parts(ring):
            if ring == "a":
                return rs_a, ag_a, sb_a, send_sems_a, recv_sems_a, right, left
            return rs_b, ag_b, sb_b, send_sems_b, recv_sems_b, left, right

        def make_send(t, s, ring):
            rs, ag, sb, send_sems, recv_sems, dst_nbr, _ = ring_parts(ring)
            src = sb.at[t, s] if t <= N_RS else ag.at[t - 1 - N_RS, s]
            dst = rs.at[t, s] if t < N_RS else ag.at[t - N_RS, s]
            return pltpu.make_async_remote_copy(
                src_ref=src, dst_ref=dst,
                send_sem=send_sems.at[t, s], recv_sem=recv_sems.at[t, s],
                device_id=(dst_nbr,), device_id_type=pl.DeviceIdType.MESH,
            )

        pending = []

        for s in range(N_SUB):
            for ring in RINGS:
                _, _, sb, *_ = ring_parts(ring)
                sb[0, s] = x_ref[rows(0, s, ring, "send"), :].astype(jnp.bfloat16)
                r = make_send(0, s, ring)
                r.start()
                pending.append(r)

        for t in range(N_HOP):
            for s in range(N_SUB):
                for ring in RINGS:
                    rs, ag, sb, _, recv_sems, _, src_nbr = ring_parts(ring)
                    rr = rows(t, s, ring, "recv")
                    dst = rs.at[t, s] if t < N_RS else ag.at[t - N_RS, s]
                    recv = pltpu.make_async_remote_copy(
                        src_ref=dst, dst_ref=dst,
                        send_sem=recv_sems.at[t, s], recv_sem=recv_sems.at[t, s],
                        device_id=(src_nbr,), device_id_type=pl.DeviceIdType.MESH,
                    )
                    recv.wait_recv()
                    if t < N_RS:
                        acc = x_ref[rr, :] + rs[t, s].astype(jnp.float32)
                        out_ref[rr, :] = acc
                        sb[t + 1, s] = acc.astype(jnp.bfloat16)
                    else:
                        out_ref[rr, :] = ag[t - N_RS, s].astype(jnp.float32)
                    if t < N_HOP - 1:
                        r = make_send(t + 1, s, ring)
                        r.start()
                        pending.append(r)

        for r in pending:
            r.wait_send()

    bf_comm = (N_RS, N_SUB, sub, n)
    return pl.pallas_call(
        body,
        out_shape=jax.ShapeDtypeStruct((m_per, n), x.dtype),
        in_specs=[pl.BlockSpec(memory_space=pltpu.VMEM)],
        out_specs=pl.BlockSpec(memory_space=pltpu.VMEM),
        scratch_shapes=[
            pltpu.VMEM(bf_comm, jnp.bfloat16),
            pltpu.VMEM(bf_comm, jnp.bfloat16),
            pltpu.VMEM(bf_comm, jnp.bfloat16),
            pltpu.VMEM(bf_comm, jnp.bfloat16),
            pltpu.VMEM((N_RS + 1, N_SUB, sub, n), jnp.bfloat16),
            pltpu.VMEM((N_RS + 1, N_SUB, sub, n), jnp.bfloat16),
            pltpu.SemaphoreType.DMA((N_HOP, N_SUB)),
            pltpu.SemaphoreType.DMA((N_HOP, N_SUB)),
            pltpu.SemaphoreType.DMA((N_HOP, N_SUB)),
            pltpu.SemaphoreType.DMA((N_HOP, N_SUB)),
        ],
        compiler_params=pltpu.CompilerParams(
            collective_id=0, vmem_limit_bytes=60 * 1024 * 1024,
        ),
    )(x)


# device time: 96322 ns/iter; 1.6609x vs baseline; 1.0084x over previous
import jax
import jax.numpy as jnp
from jax import lax
from jax.experimental import pallas as pl
from jax.experimental.pallas import tpu as pltpu

N_DEV = 4
N_RS = N_DEV - 1
N_HOP = 2 * N_RS
N_SUB = 4
RINGS = ("a", "b")


def kernel(x):
    m_per, n = x.shape
    half = m_per // 2
    chunk = half // N_DEV
    sub = chunk // N_SUB

    def send_chunk(d, t, ring):
        if t < N_RS:
            return (d - t) % N_DEV if ring == "a" else (d + t) % N_DEV
        s = t - N_RS
        return (d + 1 - s) % N_DEV if ring == "a" else (d - 1 + s) % N_DEV

    def recv_chunk(d, t, ring):
        if t < N_RS:
            return (d - t - 1) % N_DEV if ring == "a" else (d + t + 1) % N_DEV
        s = t - N_RS
        return (d - s) % N_DEV if ring == "a" else (d + s) % N_DEV

    def body(x_ref, out_ref, rs_a, rs_b, ag_a, ag_b, sb_a, sb_b,
             send_sems_a, recv_sems_a, send_sems_b, recv_sems_b):
        d = lax.axis_index("i")
        left = (d - 1) % N_DEV
        right = (d + 1) % N_DEV

        barrier_sem = pltpu.get_barrier_semaphore()
        for nbr in (left, right):
            pl.semaphore_signal(
                barrier_sem, inc=1,
                device_id=(nbr,), device_id_type=pl.DeviceIdType.MESH,
            )
        pl.semaphore_wait(barrier_sem, 2)

        def rows(t, s, ring, which):
            c = (send_chunk if which == "send" else recv_chunk)(d, t, ring)
            base = 0 if ring == "a" else half
            return pl.ds(base + c * chunk + s * sub, sub)

        def ring_parts(ring):
            if ring == "a":
                return rs_a, ag_a, sb_a, send_sems_a, recv_sems_a, right, left
            return rs_b, ag_b, sb_b, send_sems_b, recv_sems_b, left, right

        def make_send(t, s, ring):
            rs, ag, sb, send_sems, recv_sems, dst_nbr, _ = ring_parts(ring)
            src = sb.at[t, s] if t <= N_RS else ag.at[t - 1 - N_RS, s]
            dst = rs.at[t, s] if t < N_RS else ag.at[t - N_RS, s]
            return pltpu.make_async_remote_copy(
                src_ref=src, dst_ref=dst,
                send_sem=send_sems.at[t, s], recv_sem=recv_sems.at[t, s],
                device_id=(dst_nbr,), device_id_type=pl.DeviceIdType.MESH,
            )

        pending = []

        for s in range(N_SUB):
            for ring in RINGS:
                _, _, sb, *_ = ring_parts(ring)
                sb[0, s] = x_ref[rows(0, s, ring, "send"), :].astype(jnp.bfloat16)
                r = make_send(0, s, ring)
                r.start()
                pending.append(r)

        for t in range(N_HOP):
            for s in range(N_SUB):
                for ring in RINGS:
                    rs, ag, sb, _, recv_sems, _, src_nbr = ring_parts(ring)
                    rr = rows(t, s, ring, "recv")
                    dst = rs.at[t, s] if t < N_RS else ag.at[t - N_RS, s]
                    recv = pltpu.make_async_remote_copy(
                        src_ref=dst, dst_ref=dst,
                        send_sem=recv_sems.at[t, s], recv_sem=recv_sems.at[t, s],
                        device_id=(src_nbr,), device_id_type=pl.DeviceIdType.MESH,
                    )
                    recv.wait_recv()
                    if t < N_RS:
                        acc = x_ref[rr, :] + rs[t, s].astype(jnp.float32)
                        if t == N_RS - 1:
                            out_ref[rr, :] = acc
                        sb[t + 1, s] = acc.astype(jnp.bfloat16)
                    else:
                        out_ref[rr, :] = ag[t - N_RS, s].astype(jnp.float32)
                    if t < N_HOP - 1:
                        r = make_send(t + 1, s, ring)
                        r.start()
                        pending.append(r)

        for r in pending:
            r.wait_send()

    bf_comm = (N_RS, N_SUB, sub, n)
    return pl.pallas_call(
        body,
        out_shape=jax.ShapeDtypeStruct((m_per, n), x.dtype),
        in_specs=[pl.BlockSpec(memory_space=pltpu.VMEM)],
        out_specs=pl.BlockSpec(memory_space=pltpu.VMEM),
        scratch_shapes=[
            pltpu.VMEM(bf_comm, jnp.bfloat16),
            pltpu.VMEM(bf_comm, jnp.bfloat16),
            pltpu.VMEM(bf_comm, jnp.bfloat16),
            pltpu.VMEM(bf_comm, jnp.bfloat16),
            pltpu.VMEM((N_RS + 1, N_SUB, sub, n), jnp.bfloat16),
            pltpu.VMEM((N_RS + 1, N_SUB, sub, n), jnp.bfloat16),
            pltpu.SemaphoreType.DMA((N_HOP, N_SUB)),
            pltpu.SemaphoreType.DMA((N_HOP, N_SUB)),
            pltpu.SemaphoreType.DMA((N_HOP, N_SUB)),
            pltpu.SemaphoreType.DMA((N_HOP, N_SUB)),
        ],
        compiler_params=pltpu.CompilerParams(
            collective_id=0, vmem_limit_bytes=60 * 1024 * 1024,
        ),
    )(x)
